# baseline (speedup 1.0000x reference)
import functools

import jax
import jax.numpy as jnp
import numpy as np
from jax.experimental import pallas as pl
from jax.experimental.pallas import tpu as pltpu

_NBUF = 8
_LA = 4


@functools.lru_cache(maxsize=None)
def _mix_constants(bs: int):
    with jax.ensure_compile_time_eval():
        key = jax.random.key(42)
        kp, kr, kc = jax.random.split(key, 3)
        perm = jax.random.permutation(kp, bs)
        keep = jax.random.uniform(kr, (bs,)) < 1.0
        perm = jnp.where(keep, perm, jnp.arange(bs))
        coeffs = jax.random.beta(kc, 0.4, 0.4, (bs,)).astype(jnp.float32)
    return np.asarray(perm, dtype=np.int32), np.asarray(coeffs, dtype=np.float32)


@functools.lru_cache(maxsize=None)
def _schedule(bs: int):
    perm, coeffs = _mix_constants(bs)
    visited = np.zeros(bs, dtype=bool)
    order, ishead, isend = [], [], []
    for s in range(bs):
        if visited[s]:
            continue
        i = s
        first = True
        while not visited[i]:
            visited[i] = True
            order.append(i)
            ishead.append(1 if first else 0)
            isend.append(0)
            first = False
            i = int(perm[i])
        isend[-1] = 1
    order = np.asarray(order, dtype=np.int32)
    ishead = np.asarray(ishead, dtype=np.int32)
    isend = np.asarray(isend, dtype=np.int32)

    ld = np.concatenate([order, np.zeros(1 + _LA, np.int32)])
    oidx = np.concatenate([order[:1], order])
    bidx = perm[oidx]
    e = np.concatenate([np.zeros(1, np.int32), isend])
    hd = np.concatenate([ishead, np.zeros(1, np.int32)])
    cs = coeffs[oidx]
    return ld, oidx, bidx, e, hd, cs


def _make_body(bs: int):
    def _body(ld, oidx, bidx, e, hd, cs,
              xin, y2f, yaf, out, yof, zof,
              ring, obuf, head, insem, outsem):
        t = pl.program_id(0)

        @pl.when(t == 0)
        def _():
            for j in range(_LA):
                pltpu.make_async_copy(
                    xin.at[ld[j]], ring.at[j], insem.at[j]).start()

        @pl.when(t + _LA < bs)
        def _():
            slot = jax.lax.rem(t + _LA, _NBUF)
            pltpu.make_async_copy(
                xin.at[ld[t + _LA]], ring.at[slot], insem.at[slot]).start()

        @pl.when(t < bs)
        def _():
            slot = jax.lax.rem(t, _NBUF)
            pltpu.make_async_copy(
                xin.at[ld[t]], ring.at[slot], insem.at[slot]).wait()

        @pl.when(t > 0)
        def _():
            u = t - 1
            c = cs[t]
            cur = jax.lax.rem(t, _NBUF)
            prv = jax.lax.rem(u, _NBUF)
            ob = jax.lax.rem(u, 2)

            @pl.when(u >= 2)
            def _():
                pltpu.make_async_copy(
                    obuf.at[ob], out.at[oidx[t - 2]], outsem.at[ob]).wait()

            @pl.when(e[t] == 0)
            def _():
                obuf[ob] = c * ring[prv] + (1.0 - c) * ring[cur]

            @pl.when(e[t] == 1)
            def _():
                obuf[ob] = c * ring[prv] + (1.0 - c) * head[...]

            pltpu.make_async_copy(
                obuf.at[ob], out.at[oidx[t]], outsem.at[ob]).start()

            o = oidx[t]
            b = bidx[t]
            ym = c * y2f[o] + (1.0 - c) * y2f[b]
            yof[o] = ym
            zof[o] = jnp.clip(jnp.maximum(yaf[o], yaf[b]) - ym, 0.0, 1.0)

        @pl.when(hd[t] == 1)
        def _():
            head[...] = ring[jax.lax.rem(t, _NBUF)]

        @pl.when(t == bs)
        def _():
            pltpu.make_async_copy(
                obuf.at[(bs - 2) % 2], out.at[oidx[bs - 1]],
                outsem.at[(bs - 2) % 2]).wait()
            pltpu.make_async_copy(
                obuf.at[(bs - 1) % 2], out.at[oidx[bs]],
                outsem.at[(bs - 1) % 2]).wait()

    return _body


def kernel(x, y, y_aux, w):
    bs = x.shape[0]
    ld, oidx, bidx, e, hd, cs = _schedule(bs)
    n = int(np.prod(x.shape[1:]))
    assert n % 128 == 0
    r = n // 128
    xr = x.reshape(bs, r, 128)

    nc = y.shape[1]
    pad = (-(nc + 1)) % 1024
    y2 = jnp.concatenate(
        [y, w[:, None], jnp.zeros((bs, pad), jnp.float32)], axis=1)
    ncp = nc + 1 + pad
    y2r = y2.reshape(bs, ncp // 128, 128)
    yar = jnp.pad(y_aux, ((0, 0), (0, ncp - nc))).reshape(bs, ncp // 128, 128)

    def full_map(t, *scal):
        return (0, 0, 0)

    yfull = pl.BlockSpec((bs, ncp // 128, 128), full_map)

    grid_spec = pltpu.PrefetchScalarGridSpec(
        num_scalar_prefetch=6,
        grid=(bs + 1,),
        in_specs=[pl.BlockSpec(memory_space=pl.ANY), yfull, yfull],
        out_specs=[pl.BlockSpec(memory_space=pl.ANY), yfull, yfull],
        scratch_shapes=[
            pltpu.VMEM((_NBUF, r, 128), jnp.float32),
            pltpu.VMEM((2, r, 128), jnp.float32),
            pltpu.VMEM((r, 128), jnp.float32),
            pltpu.SemaphoreType.DMA((_NBUF,)),
            pltpu.SemaphoreType.DMA((2,)),
        ],
    )

    xo, yo, zo = pl.pallas_call(
        _make_body(bs),
        grid_spec=grid_spec,
        out_shape=[
            jax.ShapeDtypeStruct((bs, r, 128), jnp.float32),
            jax.ShapeDtypeStruct((bs, ncp // 128, 128), jnp.float32),
            jax.ShapeDtypeStruct((bs, ncp // 128, 128), jnp.float32),
        ],
        compiler_params=pltpu.CompilerParams(
            dimension_semantics=("arbitrary",),
        ),
    )(jnp.asarray(ld), jnp.asarray(oidx), jnp.asarray(bidx),
      jnp.asarray(e), jnp.asarray(hd), jnp.asarray(cs),
      xr, y2r, yar)

    x_mix = xo.reshape(x.shape)
    yo2 = yo.reshape(bs, ncp)
    y_mix = yo2[:, :nc]
    w_mix = yo2[:, nc]
    ya_mix = zo.reshape(bs, ncp)[:, :nc]
    return (x_mix, y_mix, ya_mix, w_mix)
